# Initial kernel scaffold; baseline (speedup 1.0000x reference)
import jax
import jax.numpy as jnp
from jax.experimental import pallas as pl


def kernel(x, table):
    raise NotImplementedError("write your pallas kernel here")


# speedup vs baseline: 2.9533x; 2.9533x over previous
import functools

import jax
import jax.numpy as jnp
from jax import lax
from jax.experimental import pallas as pl
from jax.experimental.pallas import tpu as pltpu
from jax.experimental.pallas import tpu_sc as plsc

D = 64
NC, NS = 2, 16
NW = NC * NS
CH = 1024
IR = CH // 128


@functools.partial(jax.jit, static_argnames=("n_chunks",))
def _sc_gather(table, idx2d, n_chunks):
    B = n_chunks * NW * CH
    mesh = plsc.VectorSubcoreMesh(core_axis_name="c", subcore_axis_name="s")

    @functools.partial(
        pl.kernel,
        out_type=jax.ShapeDtypeStruct((B, D), jnp.float32),
        mesh=mesh,
        scratch_types=[
            pltpu.VMEM((IR, 128), jnp.int32),
            pltpu.VMEM((CH, D), jnp.float32),
            pltpu.SemaphoreType.DMA,
        ],
        compiler_params=pltpu.CompilerParams(use_tc_tiling_on_sc=False),
    )
    def k(table_hbm, idx_hbm, out_hbm, idx_v, rows_v, sem):
        wid = lax.axis_index("s") * NC + lax.axis_index("c")

        @pl.loop(0, n_chunks)
        def _chunk(i):
            row0 = (wid * n_chunks + i) * CH
            irow0 = (wid * n_chunks + i) * IR
            pltpu.sync_copy(idx_hbm.at[pl.ds(irow0, IR)], idx_v)
            for j in range(IR):
                pltpu.async_copy(
                    table_hbm.at[idx_v.at[j]],
                    rows_v.at[pl.ds(j * 128, 128)],
                    sem,
                )
            for j in range(IR):
                pltpu.make_async_copy(
                    table_hbm.at[idx_v.at[j]],
                    rows_v.at[pl.ds(j * 128, 128)],
                    sem,
                ).wait()
            pltpu.sync_copy(rows_v, out_hbm.at[pl.ds(row0, CH)])

    return k(table, idx2d)


def kernel(x, table):
    n0, n1 = x.shape
    B = n0 * n1
    idx2d = x.reshape(B // 128, 128).astype(jnp.int32)
    out = _sc_gather(table, idx2d, B // (NW * CH))
    return out.reshape(n0, n1, D)


# speedup vs baseline: 5.2392x; 1.7740x over previous
import functools

import jax
import jax.numpy as jnp
from jax import lax
from jax.experimental import pallas as pl
from jax.experimental.pallas import tpu as pltpu
from jax.experimental.pallas import tpu_sc as plsc

D = 64
NC, NS = 2, 16
NW = NC * NS
CH = 1024
IR = CH // 128


@functools.partial(jax.jit, static_argnames=("n_chunks",))
def _sc_gather(table, idx2d, n_chunks):
    B = n_chunks * NW * CH
    mesh = plsc.VectorSubcoreMesh(core_axis_name="c", subcore_axis_name="s")

    @functools.partial(
        pl.kernel,
        out_type=jax.ShapeDtypeStruct((B, D), jnp.float32),
        mesh=mesh,
        scratch_types=[
            pltpu.VMEM((IR, 128), jnp.int32),
            pltpu.VMEM((CH, D), jnp.float32),
            pltpu.VMEM_SHARED((169, D), jnp.float32),
            pltpu.SemaphoreType.DMA,
        ],
        compiler_params=pltpu.CompilerParams(use_tc_tiling_on_sc=False),
    )
    def k(table_hbm, idx_hbm, out_hbm, idx_v, rows_v, table_s, sem):
        wid = lax.axis_index("s") * NC + lax.axis_index("c")
        sid = lax.axis_index("s")

        @pl.when(sid == 0)
        def _stage_table():
            pltpu.sync_copy(table_hbm, table_s)

        plsc.subcore_barrier()

        @pl.loop(0, n_chunks)
        def _chunk(i):
            row0 = (wid * n_chunks + i) * CH
            irow0 = (wid * n_chunks + i) * IR
            pltpu.sync_copy(idx_hbm.at[pl.ds(irow0, IR)], idx_v)
            for j in range(IR):
                pltpu.async_copy(
                    table_s.at[idx_v.at[j]],
                    rows_v.at[pl.ds(j * 128, 128)],
                    sem,
                )
            for j in range(IR):
                pltpu.make_async_copy(
                    table_s.at[idx_v.at[j]],
                    rows_v.at[pl.ds(j * 128, 128)],
                    sem,
                ).wait()
            pltpu.sync_copy(rows_v, out_hbm.at[pl.ds(row0, CH)])

    return k(table, idx2d)


def kernel(x, table):
    n0, n1 = x.shape
    B = n0 * n1
    idx2d = x.reshape(B // 128, 128).astype(jnp.int32)
    out = _sc_gather(table, idx2d, B // (NW * CH))
    return out.reshape(n0, n1, D)


# speedup vs baseline: 5.8098x; 1.1089x over previous
import functools

import jax
import jax.numpy as jnp
from jax import lax
from jax.experimental import pallas as pl
from jax.experimental.pallas import tpu as pltpu
from jax.experimental.pallas import tpu_sc as plsc

D = 64
NC, NS = 2, 16
NW = NC * NS
CH = 640
IR = CH // 128


@functools.partial(jax.jit, static_argnames=("n_chunks",))
def _sc_gather(table, idx2d, n_chunks):
    B = n_chunks * NW * CH
    mesh = plsc.VectorSubcoreMesh(core_axis_name="c", subcore_axis_name="s")

    @functools.partial(
        pl.kernel,
        out_type=jax.ShapeDtypeStruct((B, D), jnp.float32),
        mesh=mesh,
        scratch_types=[
            pltpu.VMEM((2, IR, 128), jnp.int32),
            pltpu.VMEM((CH, D), jnp.float32),
            pltpu.VMEM((CH, D), jnp.float32),
            pltpu.VMEM_SHARED((169, D), jnp.float32),
            pltpu.SemaphoreType.DMA,
            pltpu.SemaphoreType.DMA,
            pltpu.SemaphoreType.DMA,
            pltpu.SemaphoreType.DMA,
            pltpu.SemaphoreType.DMA,
        ],
        compiler_params=pltpu.CompilerParams(use_tc_tiling_on_sc=False),
    )
    def k(table_hbm, idx_hbm, out_hbm, idx_v, rows0, rows1, table_s,
          gat_sem, idx_sem0, idx_sem1, out_sem0, out_sem1):
        rows_v = (rows0, rows1)
        idx_sem = (idx_sem0, idx_sem1)
        out_sem = (out_sem0, out_sem1)
        wid = lax.axis_index("s") * NC + lax.axis_index("c")
        sid = lax.axis_index("s")

        @pl.when(sid == 0)
        def _stage_table():
            pltpu.sync_copy(table_hbm, table_s)

        plsc.subcore_barrier()

        def irow0(i):
            return (wid * n_chunks + i) * IR

        def fire_idx(i, b):
            pltpu.async_copy(
                idx_hbm.at[pl.ds(irow0(i), IR)], idx_v.at[b], idx_sem[b])

        fire_idx(0, 0)
        fire_idx(1, 1)

        @pl.loop(0, n_chunks, step=2)
        def _chunk(g):
            for b in range(2):
                i = g + b
                pltpu.make_async_copy(
                    idx_hbm.at[pl.ds(irow0(i), IR)], idx_v.at[b],
                    idx_sem[b]).wait()

                @pl.when(g >= 2)
                def _drain_store():
                    pltpu.make_async_copy(
                        rows_v[b],
                        out_hbm.at[pl.ds((wid * n_chunks + i - 2) * CH, CH)],
                        out_sem[b]).wait()

                for j in range(IR):
                    pltpu.async_copy(
                        table_s.at[idx_v.at[b].at[j]],
                        rows_v[b].at[pl.ds(j * 128, 128)],
                        gat_sem,
                    )
                for j in range(IR):
                    pltpu.make_async_copy(
                        table_s.at[idx_v.at[b].at[j]],
                        rows_v[b].at[pl.ds(j * 128, 128)],
                        gat_sem,
                    ).wait()

                @pl.when(i + 2 < n_chunks)
                def _prefetch_idx():
                    fire_idx(i + 2, b)

                pltpu.async_copy(
                    rows_v[b],
                    out_hbm.at[pl.ds((wid * n_chunks + i) * CH, CH)],
                    out_sem[b])

        for b in range(2):
            i = n_chunks - 2 + b
            pltpu.make_async_copy(
                rows_v[b],
                out_hbm.at[pl.ds((wid * n_chunks + i) * CH, CH)],
                out_sem[b]).wait()

    return k(table, idx2d)


def kernel(x, table):
    n0, n1 = x.shape
    B = n0 * n1
    idx2d = x.reshape(B // 128, 128).astype(jnp.int32)
    out = _sc_gather(table, idx2d, B // (NW * CH))
    return out.reshape(n0, n1, D)
